# baseline (speedup 1.0000x reference)
import dataclasses
import functools

import jax
import jax.numpy as jnp
from jax import lax
from jax.experimental import pallas as pl
from jax.experimental.pallas import tpu as pltpu
from jax.experimental.pallas import tpu_sc as plsc

LANES = 16
R = 4
NB_IN = 4
NB_OUT = 2
NW = 32


def kernel(x, permutation):
    batch, dim = x.shape
    perm = permutation.astype(jnp.int32)
    rows_per_w = batch // NW
    n_chunks = rows_per_w // R

    mesh = plsc.VectorSubcoreMesh(core_axis_name="c", subcore_axis_name="s")

    cp = pltpu.CompilerParams()
    if "needs_layout_passes" in pltpu.CompilerParams.__dataclass_fields__:
        cp = dataclasses.replace(cp, needs_layout_passes=False)

    @functools.partial(
        pl.kernel,
        out_type=jax.ShapeDtypeStruct((batch, dim), x.dtype),
        mesh=mesh,
        scratch_types=[
            pltpu.VMEM((dim,), jnp.int32),
            pltpu.VMEM((NB_IN, R, dim), jnp.float32),
            pltpu.VMEM((NB_OUT, R, dim), jnp.float32),
            pltpu.SemaphoreType.DMA((NB_IN,)),
            pltpu.SemaphoreType.DMA((NB_OUT,)),
        ],
        compiler_params=cp,
    )
    def permute_kernel(x_hbm, perm_hbm, out_hbm, perm_v, in_v, out_v,
                       sem_in, sem_out):
        wid = lax.axis_index("s") * 2 + lax.axis_index("c")
        base = wid * rows_per_w
        pltpu.sync_copy(perm_hbm, perm_v)

        def start_in(g, b):
            pltpu.make_async_copy(
                x_hbm.at[pl.ds(base + g * R, R)], in_v.at[b],
                sem_in.at[b]).start()

        def wait_in(b):
            pltpu.make_async_copy(
                x_hbm.at[pl.ds(base, R)], in_v.at[b], sem_in.at[b]).wait()

        def start_out(g, h):
            pltpu.make_async_copy(
                out_v.at[h], out_hbm.at[pl.ds(base + g * R, R)],
                sem_out.at[h]).start()

        def wait_out(h):
            pltpu.make_async_copy(
                out_v.at[h], out_hbm.at[pl.ds(base, R)], sem_out.at[h]).wait()

        for p in range(NB_IN - 1):
            start_in(p, p)

        @pl.loop(0, n_chunks // NB_IN)
        def _(k):
            for b in range(NB_IN):
                g = NB_IN * k + b
                wait_in(b)

                @pl.when(g + NB_IN - 1 < n_chunks)
                def _():
                    start_in(g + NB_IN - 1, (b + NB_IN - 1) % NB_IN)

                ho = b % NB_OUT

                @pl.when(g > NB_OUT - 1)
                def _():
                    wait_out(ho)

                @plsc.parallel_loop(0, dim, step=LANES, unroll=16)
                def _(c):
                    col = pl.ds(c, LANES)
                    idx = perm_v[col]
                    for r in range(R):
                        row = jnp.full((LANES,), r, jnp.int32)
                        out_v[ho, r, col] = plsc.load_gather(
                            in_v.at[b], [row, idx])

                start_out(g, ho)

        wait_out(0)
        wait_out(1)

    return permute_kernel(x, perm)


# speedup vs baseline: 1.0016x; 1.0016x over previous
import dataclasses
import functools

import jax
import jax.numpy as jnp
from jax import lax
from jax.experimental import pallas as pl
from jax.experimental.pallas import tpu as pltpu
from jax.experimental.pallas import tpu_sc as plsc

LANES = 16
R = 4
NB_IN = 4
NB_OUT = 2
NW = 32


def kernel(x, permutation):
    batch, dim = x.shape
    perm = permutation.astype(jnp.int32)
    rows_per_w = batch // NW
    n_chunks = rows_per_w // R

    mesh = plsc.VectorSubcoreMesh(core_axis_name="c", subcore_axis_name="s")

    cp = pltpu.CompilerParams()
    if "needs_layout_passes" in pltpu.CompilerParams.__dataclass_fields__:
        cp = dataclasses.replace(cp, needs_layout_passes=False)

    @functools.partial(
        pl.kernel,
        out_type=jax.ShapeDtypeStruct((batch, dim), x.dtype),
        mesh=mesh,
        scratch_types=[
            pltpu.VMEM((dim,), jnp.int32),
            pltpu.VMEM((NB_IN, R, dim), jnp.float32),
            pltpu.VMEM((NB_OUT, R, dim), jnp.float32),
            pltpu.SemaphoreType.DMA((NB_IN,)),
            pltpu.SemaphoreType.DMA((NB_OUT,)),
        ],
        compiler_params=cp,
    )
    def permute_kernel(x_hbm, perm_hbm, out_hbm, perm_v, in_v, out_v,
                       sem_in, sem_out):
        wid = lax.axis_index("s") * 2 + lax.axis_index("c")
        base = wid * rows_per_w
        def start_in(g, b):
            pltpu.make_async_copy(
                x_hbm.at[pl.ds(base + g * R, R)], in_v.at[b],
                sem_in.at[b]).start()

        def wait_in(b):
            pltpu.make_async_copy(
                x_hbm.at[pl.ds(base, R)], in_v.at[b], sem_in.at[b]).wait()

        def start_out(g, h):
            pltpu.make_async_copy(
                out_v.at[h], out_hbm.at[pl.ds(base + g * R, R)],
                sem_out.at[h]).start()

        def wait_out(h):
            pltpu.make_async_copy(
                out_v.at[h], out_hbm.at[pl.ds(base, R)], sem_out.at[h]).wait()

        for p in range(NB_IN - 1):
            start_in(p, p)
        pltpu.sync_copy(perm_hbm, perm_v)

        @pl.loop(0, n_chunks // NB_IN)
        def _(k):
            for b in range(NB_IN):
                g = NB_IN * k + b
                wait_in(b)

                @pl.when(g + NB_IN - 1 < n_chunks)
                def _():
                    start_in(g + NB_IN - 1, (b + NB_IN - 1) % NB_IN)

                ho = b % NB_OUT

                @pl.when(g > NB_OUT - 1)
                def _():
                    wait_out(ho)

                @plsc.parallel_loop(0, dim, step=LANES, unroll=16)
                def _(c):
                    col = pl.ds(c, LANES)
                    idx = perm_v[col]
                    for r in range(R):
                        row = jnp.full((LANES,), r, jnp.int32)
                        out_v[ho, r, col] = plsc.load_gather(
                            in_v.at[b], [row, idx])

                start_out(g, ho)

        wait_out(0)
        wait_out(1)

    return permute_kernel(x, perm)
